# baseline (speedup 1.0000x reference)
import functools

import jax
import jax.numpy as jnp
from jax import lax
from jax.experimental import pallas as pl
from jax.experimental.pallas import tpu as pltpu
from jax.experimental.pallas import tpu_sc as plsc

NC = 2
NS = 16
NW = NC * NS
L = 16


def _make_deg_kernel(EP, NP):
    EV = EP // (NW * L)
    SPT = NP // NS
    mesh = plsc.VectorSubcoreMesh(core_axis_name="c", subcore_axis_name="s")

    @functools.partial(
        pl.kernel,
        mesh=mesh,
        out_type=jax.ShapeDtypeStruct((NC, NP), jnp.float32),
        scratch_types=[
            pltpu.VMEM((EP // NW,), jnp.int32),
            pltpu.VMEM((NP,), jnp.float32),
            pltpu.VMEM((SPT,), jnp.float32),
            pltpu.VMEM((SPT,), jnp.float32),
            pltpu.VMEM_SHARED((NS, NP), jnp.float32),
        ],
        compiler_params=pltpu.CompilerParams(needs_layout_passes=False),
    )
    def deg_kernel(dst_hbm, out_hbm, idx_v, hist_v, acc_v, bounce_v, hist_sh):
        cid = lax.axis_index("c")
        sid = lax.axis_index("s")
        wid = sid * NC + cid

        zeros = jnp.zeros((L,), jnp.float32)
        ones = jnp.ones((L,), jnp.float32)

        def zero_hist(i, _):
            hist_v[pl.ds(i * L, L)] = zeros
            return _
        lax.fori_loop(0, NP // L, zero_hist, None)

        pltpu.sync_copy(dst_hbm.at[pl.ds(wid * (EP // NW), EP // NW)], idx_v)

        def count(j, _):
            idx = idx_v[pl.ds(j * L, L)]
            plsc.addupdate_scatter(hist_v, [idx], ones)
            return _
        lax.fori_loop(0, EV, count, None)

        pltpu.sync_copy(hist_v, hist_sh.at[sid])
        plsc.subcore_barrier()

        pltpu.sync_copy(hist_sh.at[0].at[pl.ds(sid * SPT, SPT)], acc_v)
        for t in range(1, NS):
            pltpu.sync_copy(hist_sh.at[t].at[pl.ds(sid * SPT, SPT)], bounce_v)

            def accum(i, _):
                s = pl.ds(i * L, L)
                acc_v[s] = acc_v[s] + bounce_v[s]
                return _
            lax.fori_loop(0, SPT // L, accum, None)

        pltpu.sync_copy(acc_v, out_hbm.at[cid].at[pl.ds(sid * SPT, SPT)])

    return deg_kernel


def _make_edge_kernel(N, NP, D, N0, N1, IB, CH):
    RPT = NP // NS
    mesh = plsc.VectorSubcoreMesh(core_axis_name="c", subcore_axis_name="s")
    assert N0 % 2 == 0 and N1 % 2 == 0 and IB % 4 == 0

    @functools.partial(
        pl.kernel,
        mesh=mesh,
        out_type=jax.ShapeDtypeStruct((NC, NP, D), jnp.float32),
        scratch_types=[
            pltpu.VMEM((2, 2 * IB, CH), jnp.int32),
            pltpu.VMEM((4, CH, D), jnp.float32),
            pltpu.VMEM_SHARED((NP, D), jnp.float32),
            pltpu.SemaphoreType.DMA((4,)),
            pltpu.SemaphoreType.DMA((4,)),
            pltpu.SemaphoreType.DMA((2,)),
        ],
        compiler_params=pltpu.CompilerParams(needs_layout_passes=False),
    )
    def edge_kernel(g_hbm, edges_hbm, out_hbm,
                    eidx_v, rows_v, acc_sh, gsem, ssem, isem):
        cid = lax.axis_index("c")
        sid = lax.axis_index("s")
        nblk = jnp.where(cid == 0, N0, N1)
        blk0 = jnp.where(cid == 0, sid * N0, NS * N0 + sid * N1)

        zeros = jnp.zeros((L,), jnp.float32)
        for k in range(D // L):
            def zrow(i, _, k=k):
                rows_v[0, i, pl.ds(k * L, L)] = zeros
                return _
            lax.fori_loop(0, CH, zrow, None)
        full, rem = RPT // CH, RPT % CH
        for t in range(full):
            pltpu.sync_copy(rows_v.at[0],
                            acc_sh.at[pl.ds(sid * RPT + t * CH, CH)])
        if rem:
            pltpu.sync_copy(rows_v.at[0].at[pl.ds(0, rem)],
                            acc_sh.at[pl.ds(sid * RPT + full * CH, rem)])

        pltpu.sync_copy(edges_hbm.at[blk0], eidx_v.at[0])
        pltpu.async_copy(g_hbm.at[eidx_v.at[0, 0]], rows_v.at[0], gsem.at[0])
        pltpu.async_copy(g_hbm.at[eidx_v.at[0, 1]], rows_v.at[1], gsem.at[1])
        plsc.subcore_barrier()

        def pair(ii, _):
            for q in range(2):
                jj = 2 * ii + q
                nb = nblk
                for b in range(IB):
                    r = b & 3
                    r2 = (b + 2) & 3
                    if b >= 2:
                        dref = acc_sh.at[eidx_v.at[q, IB + b - 2]]
                        pltpu.make_async_copy(rows_v.at[r2], dref,
                                              ssem.at[r2]).wait()
                    else:
                        def _wait_prev(q=q, b=b, r2=r2):
                            dref = acc_sh.at[eidx_v.at[1 - q, 2 * IB - 2 + b]]
                            pltpu.make_async_copy(rows_v.at[r2], dref,
                                                  ssem.at[r2]).wait()
                        if q == 1:
                            _wait_prev()
                        else:
                            pl.when(ii > 0)(_wait_prev)
                    if b == 2:
                        @pl.when(jj + 1 < nb)
                        def _prefetch_idx(q=q, jj=jj):
                            pltpu.async_copy(edges_hbm.at[blk0 + jj + 1],
                                             eidx_v.at[1 - q],
                                             isem.at[1 - q])
                    if b + 2 < IB:
                        pltpu.async_copy(g_hbm.at[eidx_v.at[q, b + 2]],
                                         rows_v.at[r2], gsem.at[r2])
                    else:
                        @pl.when(jj + 1 < nb)
                        def _cross_gather(q=q, b=b, r2=r2):
                            if b == IB - 2:
                                pltpu.make_async_copy(
                                    edges_hbm.at[blk0], eidx_v.at[1 - q],
                                    isem.at[1 - q]).wait()
                            pltpu.async_copy(
                                g_hbm.at[eidx_v.at[1 - q, b + 2 - IB]],
                                rows_v.at[r2], gsem.at[r2])
                    pltpu.make_async_copy(g_hbm.at[eidx_v.at[q, b]],
                                          rows_v.at[r], gsem.at[r]).wait()
                    pltpu.async_copy(rows_v.at[r],
                                     acc_sh.at[eidx_v.at[q, IB + b]],
                                     ssem.at[r], add=True)
            return _
        lax.fori_loop(0, nblk // 2, pair, None)

        for b in (IB - 2, IB - 1):
            pltpu.make_async_copy(rows_v.at[b & 3],
                                  acc_sh.at[eidx_v.at[1, IB + b]],
                                  ssem.at[b & 3]).wait()
        plsc.subcore_barrier()

        pltpu.sync_copy(acc_sh.at[pl.ds(sid * RPT, RPT)],
                        out_hbm.at[cid].at[pl.ds(sid * RPT, RPT)])

    return edge_kernel


def _tc1_body(x_ref, w_ref, da_ref, db_ref, g_ref, dinv_ref):
    deg = da_ref[...] + db_ref[...] + 1.0
    dinv = lax.rsqrt(deg)
    h = jnp.dot(x_ref[...], w_ref[...], preferred_element_type=jnp.float32)
    g_ref[...] = h * dinv
    dinv_ref[...] = dinv


def _tc2_body(acc_ref, g_ref, dinv_ref, b_ref, w_ref, g2_ref):
    n = g_ref.shape[0]
    dinv = dinv_ref[...]
    acc = acc_ref[0, :n, :] + acc_ref[1, :n, :] + g_ref[...]
    h = jnp.maximum(dinv * acc + b_ref[...], 0.0)
    h2 = jnp.dot(h, w_ref[...], preferred_element_type=jnp.float32)
    g2_ref[...] = h2 * dinv


def _tc3_body(acc_ref, g_ref, dinv_ref, b_ref, out_ref):
    n = g_ref.shape[0]
    acc = acc_ref[0, :n, :] + acc_ref[1, :n, :] + g_ref[...]
    out_ref[...] = dinv_ref[...] * acc + b_ref[...]


def kernel(x, edge_index, W1, b1, W2, b2):
    N, D_IN = x.shape
    D_HID = W1.shape[1]
    D_OUT = W2.shape[1]
    E = edge_index.shape[1]

    NP = -(-(N + 1) // 128) * 128
    NPD = -(-(N + 1) // 2048) * 2048
    IB = 8
    CH = 64
    NBLK = -(-E // (NW * IB * CH * 2)) * 2
    SPLIT0 = 8
    N0, N1 = SPLIT0, 2 * NBLK - SPLIT0
    EP = NW * NBLK * IB * CH
    assert NP % NS == 0

    src = edge_index[0]
    dst = edge_index[1]
    pad = EP - E
    srcp = jnp.concatenate([src, jnp.zeros((pad,), jnp.int32)])
    dstp = jnp.concatenate([dst, jnp.full((pad,), N, jnp.int32)])
    s4 = srcp.reshape(NW * NBLK, IB, CH)
    d4 = dstp.reshape(NW * NBLK, IB, CH)
    edges = jnp.concatenate([s4, d4], axis=1)

    deg_kernel = _make_deg_kernel(EP, NPD)
    edge_kernel = _make_edge_kernel(N, NP, D_HID, N0, N1, IB, CH)

    degp = deg_kernel(dstp)
    da = degp[0].reshape(NPD, 1)[:N]
    db = degp[1].reshape(NPD, 1)[:N]

    g1, dinv = pl.pallas_call(
        _tc1_body,
        out_shape=(
            jax.ShapeDtypeStruct((N, D_HID), jnp.float32),
            jax.ShapeDtypeStruct((N, 1), jnp.float32),
        ),
    )(x, W1, da, db)

    acc1 = edge_kernel(g1, edges)

    g2 = pl.pallas_call(
        _tc2_body,
        out_shape=jax.ShapeDtypeStruct((N, D_HID), jnp.float32),
    )(acc1, g1, dinv, b1.reshape(1, D_HID), W2)

    acc2 = edge_kernel(g2, edges)

    out = pl.pallas_call(
        _tc3_body,
        out_shape=jax.ShapeDtypeStruct((N, D_OUT), jnp.float32),
    )(acc2, g2, dinv, b2.reshape(1, D_OUT))
    return out


# speedup vs baseline: 1.0902x; 1.0902x over previous
import functools

import jax
import jax.numpy as jnp
from jax import lax
from jax.experimental import pallas as pl
from jax.experimental.pallas import tpu as pltpu
from jax.experimental.pallas import tpu_sc as plsc

NC = 2
NS = 16
NW = NC * NS
L = 16


def _make_deg_kernel(EP, NP):
    EV = EP // (NW * L)
    SPT = NP // NS
    mesh = plsc.VectorSubcoreMesh(core_axis_name="c", subcore_axis_name="s")

    @functools.partial(
        pl.kernel,
        mesh=mesh,
        out_type=jax.ShapeDtypeStruct((NC, NP), jnp.float32),
        scratch_types=[
            pltpu.VMEM((EP // NW,), jnp.int32),
            pltpu.VMEM((NP,), jnp.float32),
            pltpu.VMEM((SPT,), jnp.float32),
            pltpu.VMEM((SPT,), jnp.float32),
            pltpu.VMEM_SHARED((NS, NP), jnp.float32),
        ],
        compiler_params=pltpu.CompilerParams(needs_layout_passes=False),
    )
    def deg_kernel(dst_hbm, out_hbm, idx_v, hist_v, acc_v, bounce_v, hist_sh):
        cid = lax.axis_index("c")
        sid = lax.axis_index("s")
        wid = sid * NC + cid

        zeros = jnp.zeros((L,), jnp.float32)
        ones = jnp.ones((L,), jnp.float32)

        def zero_hist(i, _):
            hist_v[pl.ds(i * L, L)] = zeros
            return _
        lax.fori_loop(0, NP // L, zero_hist, None)

        pltpu.sync_copy(dst_hbm.at[pl.ds(wid * (EP // NW), EP // NW)], idx_v)

        def count(j, _):
            idx = idx_v[pl.ds(j * L, L)]
            plsc.addupdate_scatter(hist_v, [idx], ones)
            return _
        lax.fori_loop(0, EV, count, None)

        pltpu.sync_copy(hist_v, hist_sh.at[sid])
        plsc.subcore_barrier()

        pltpu.sync_copy(hist_sh.at[0].at[pl.ds(sid * SPT, SPT)], acc_v)
        for t in range(1, NS):
            pltpu.sync_copy(hist_sh.at[t].at[pl.ds(sid * SPT, SPT)], bounce_v)

            def accum(i, _):
                s = pl.ds(i * L, L)
                acc_v[s] = acc_v[s] + bounce_v[s]
                return _
            lax.fori_loop(0, SPT // L, accum, None)

        pltpu.sync_copy(acc_v, out_hbm.at[cid].at[pl.ds(sid * SPT, SPT)])

    return deg_kernel


def _make_edge_kernel(N, NP, D, N0, N1, IB, CH):
    RPT = NP // NS
    mesh = plsc.VectorSubcoreMesh(core_axis_name="c", subcore_axis_name="s")
    assert N0 % 2 == 0 and N1 % 2 == 0 and IB % 4 == 0

    @functools.partial(
        pl.kernel,
        mesh=mesh,
        out_type=jax.ShapeDtypeStruct((NC, NP, D), jnp.float32),
        scratch_types=[
            pltpu.VMEM((2, 2 * IB, CH), jnp.int32),
            pltpu.VMEM((4, CH, D), jnp.float32),
            pltpu.VMEM_SHARED((NP, D), jnp.float32),
            pltpu.SemaphoreType.DMA((4,)),
            pltpu.SemaphoreType.DMA((4,)),
            pltpu.SemaphoreType.DMA((2,)),
        ],
        compiler_params=pltpu.CompilerParams(needs_layout_passes=False),
    )
    def edge_kernel(g_hbm, edges_hbm, out_hbm,
                    eidx_v, rows_v, acc_sh, gsem, ssem, isem):
        cid = lax.axis_index("c")
        sid = lax.axis_index("s")
        nblk = jnp.where(cid == 0, N0, N1)
        blk0 = jnp.where(cid == 0, sid * N0, NS * N0 + sid * N1)

        zeros = jnp.zeros((L,), jnp.float32)
        for k in range(D // L):
            def zrow(i, _, k=k):
                rows_v[0, i, pl.ds(k * L, L)] = zeros
                return _
            lax.fori_loop(0, CH, zrow, None)
        full, rem = RPT // CH, RPT % CH
        for t in range(full):
            pltpu.sync_copy(rows_v.at[0],
                            acc_sh.at[pl.ds(sid * RPT + t * CH, CH)])
        if rem:
            pltpu.sync_copy(rows_v.at[0].at[pl.ds(0, rem)],
                            acc_sh.at[pl.ds(sid * RPT + full * CH, rem)])

        pltpu.sync_copy(edges_hbm.at[blk0], eidx_v.at[0])
        pltpu.async_copy(g_hbm.at[eidx_v.at[0, 0]], rows_v.at[0], gsem.at[0])
        pltpu.async_copy(g_hbm.at[eidx_v.at[0, 1]], rows_v.at[1], gsem.at[1])
        plsc.subcore_barrier()

        def pair(ii, _):
            for q in range(2):
                jj = 2 * ii + q
                for b in range(IB):
                    r = b & 3
                    r2 = (b + 2) & 3
                    if b >= 2:
                        dref = acc_sh.at[eidx_v.at[q, IB + b - 2]]
                        pltpu.make_async_copy(rows_v.at[r2], dref,
                                              ssem.at[r2]).wait()
                    else:
                        def _wait_prev(q=q, b=b, r2=r2):
                            dref = acc_sh.at[eidx_v.at[1 - q, 2 * IB - 2 + b]]
                            pltpu.make_async_copy(rows_v.at[r2], dref,
                                                  ssem.at[r2]).wait()
                        if q == 1:
                            _wait_prev()
                        else:
                            pl.when(ii > 0)(_wait_prev)
                    if b == 2:
                        @pl.when(jj + 1 < nblk)
                        def _prefetch_idx(q=q, jj=jj):
                            pltpu.async_copy(edges_hbm.at[blk0 + jj + 1],
                                             eidx_v.at[1 - q],
                                             isem.at[1 - q])
                    if b + 2 < IB:
                        pltpu.async_copy(g_hbm.at[eidx_v.at[q, b + 2]],
                                         rows_v.at[r2], gsem.at[r2])
                    else:
                        @pl.when(jj + 1 < nblk)
                        def _cross_gather(q=q, b=b, r2=r2):
                            if b == IB - 2:
                                pltpu.make_async_copy(
                                    edges_hbm.at[blk0], eidx_v.at[1 - q],
                                    isem.at[1 - q]).wait()
                            pltpu.async_copy(
                                g_hbm.at[eidx_v.at[1 - q, b + 2 - IB]],
                                rows_v.at[r2], gsem.at[r2])
                    pltpu.make_async_copy(g_hbm.at[eidx_v.at[q, b]],
                                          rows_v.at[r], gsem.at[r]).wait()
                    pltpu.async_copy(rows_v.at[r],
                                     acc_sh.at[eidx_v.at[q, IB + b]],
                                     ssem.at[r], add=True)
            return _
        lax.fori_loop(0, nblk // 2, pair, None)

        for b in (IB - 2, IB - 1):
            pltpu.make_async_copy(rows_v.at[b & 3],
                                  acc_sh.at[eidx_v.at[1, IB + b]],
                                  ssem.at[b & 3]).wait()
        plsc.subcore_barrier()

        pltpu.sync_copy(acc_sh.at[pl.ds(sid * RPT, RPT)],
                        out_hbm.at[cid].at[pl.ds(sid * RPT, RPT)])

    return edge_kernel


def _tc1_body(x_ref, w_ref, da_ref, db_ref, g_ref, dinv_ref):
    deg = da_ref[...] + db_ref[...] + 1.0
    dinv = lax.rsqrt(deg)
    h = jnp.dot(x_ref[...], w_ref[...], preferred_element_type=jnp.float32)
    g_ref[...] = h * dinv
    dinv_ref[...] = dinv


def _tc2_body(acc_ref, g_ref, dinv_ref, b_ref, w_ref, g2_ref):
    n = g_ref.shape[0]
    dinv = dinv_ref[...]
    acc = acc_ref[0, :n, :] + acc_ref[1, :n, :] + g_ref[...]
    h = jnp.maximum(dinv * acc + b_ref[...], 0.0)
    h2 = jnp.dot(h, w_ref[...], preferred_element_type=jnp.float32)
    g2_ref[...] = h2 * dinv


def _tc3_body(acc_ref, g_ref, dinv_ref, b_ref, out_ref):
    n = g_ref.shape[0]
    acc = acc_ref[0, :n, :] + acc_ref[1, :n, :] + g_ref[...]
    out_ref[...] = dinv_ref[...] * acc + b_ref[...]


def kernel(x, edge_index, W1, b1, W2, b2):
    N, D_IN = x.shape
    D_HID = W1.shape[1]
    D_OUT = W2.shape[1]
    E = edge_index.shape[1]

    NP = -(-(N + 1) // 128) * 128
    NPD = -(-(N + 1) // 2048) * 2048
    IB = 8
    CH = 64
    NBLK = -(-E // (NW * IB * CH * 2)) * 2
    SPLIT0 = 32
    N0, N1 = SPLIT0, 2 * NBLK - SPLIT0
    EP = NW * NBLK * IB * CH
    assert NP % NS == 0

    src = edge_index[0]
    dst = edge_index[1]
    pad = EP - E
    srcp = jnp.concatenate([src, jnp.zeros((pad,), jnp.int32)])
    dstp = jnp.concatenate([dst, jnp.full((pad,), N, jnp.int32)])
    s4 = srcp.reshape(NW * NBLK, IB, CH)
    d4 = dstp.reshape(NW * NBLK, IB, CH)
    edges = jnp.concatenate([s4, d4], axis=1)

    deg_kernel = _make_deg_kernel(EP, NPD)
    edge_kernel = _make_edge_kernel(N, NP, D_HID, N0, N1, IB, CH)

    degp = deg_kernel(dstp)
    da = degp[0].reshape(NPD, 1)[:N]
    db = degp[1].reshape(NPD, 1)[:N]

    g1, dinv = pl.pallas_call(
        _tc1_body,
        out_shape=(
            jax.ShapeDtypeStruct((N, D_HID), jnp.float32),
            jax.ShapeDtypeStruct((N, 1), jnp.float32),
        ),
    )(x, W1, da, db)

    acc1 = edge_kernel(g1, edges)

    g2 = pl.pallas_call(
        _tc2_body,
        out_shape=jax.ShapeDtypeStruct((N, D_HID), jnp.float32),
    )(acc1, g1, dinv, b1.reshape(1, D_HID), W2)

    acc2 = edge_kernel(g2, edges)

    out = pl.pallas_call(
        _tc3_body,
        out_shape=jax.ShapeDtypeStruct((N, D_OUT), jnp.float32),
    )(acc2, g2, dinv, b2.reshape(1, D_OUT))
    return out
